# baseline (speedup 1.0000x reference)
import jax
import jax.numpy as jnp
from jax.experimental import pallas as pl


def _body(x_ref, m_ref, o_ref):
    o_ref[...] = x_ref[...] * m_ref[...]


def kernel(input_tensor, mask_tensor):
    B, T, F = input_tensor.shape
    N = B * T
    x = input_tensor.reshape(N, F)
    m = mask_tensor.reshape(N, 1)
    R = 1024
    out = pl.pallas_call(
        _body,
        grid=(N // R,),
        in_specs=[
            pl.BlockSpec((R, F), lambda i: (i, 0)),
            pl.BlockSpec((R, 1), lambda i: (i, 0)),
        ],
        out_specs=pl.BlockSpec((R, F), lambda i: (i, 0)),
        out_shape=jax.ShapeDtypeStruct((N, F), x.dtype),
    )(x, m)
    return out.reshape(B, T, F)
